# baseline (speedup 1.0000x reference)
import jax
import jax.numpy as jnp
from jax.experimental import pallas as pl
from jax.experimental.pallas import tpu as pltpu

_B, _S, _D = 4, 2048, 2048
_E = 8
_TOPK = 2
_HID = 8192
_SUB = _D // _E
_SUBH = _HID // _E

_RK = 16384
_SBLK = 512


def _gelu(x):
    return x * 0.5 * (1.0 + jax.lax.erf(x * 0.7071067811865476))


def _router_body(xf_ref, w_ref, bsw_ref, idx_ref, acc_ref):
    i = pl.program_id(0)

    @pl.when(i == 0)
    def _init():
        acc_ref[...] = jnp.zeros_like(acc_ref)

    acc_ref[...] += jnp.dot(
        xf_ref[...], w_ref[...],
        preferred_element_type=jnp.float32,
        precision=jax.lax.Precision.HIGHEST,
    )

    @pl.when(i == pl.num_programs(0) - 1)
    def _finish():
        logits = acc_ref[...] + bsw_ref[...]
        ids = jax.lax.broadcasted_iota(jnp.int32, logits.shape, 1)
        top1 = jnp.argmax(logits, axis=1).astype(jnp.int32)
        masked = jnp.where(ids == top1[:, None], -jnp.inf, logits)
        top2 = jnp.argmax(masked, axis=1).astype(jnp.int32)
        idx_ref[...] = jnp.stack([top1, top2], axis=1)


def _route(x_flat, W_switch, b_switch):
    nsteps = (_S * _D) // _RK
    return pl.pallas_call(
        _router_body,
        grid=(nsteps,),
        in_specs=[
            pl.BlockSpec((_B, _RK), lambda i: (0, i)),
            pl.BlockSpec((_RK, _E), lambda i: (i, 0)),
            pl.BlockSpec((1, _E), lambda i: (0, 0)),
        ],
        out_specs=pl.BlockSpec((_B, _TOPK), lambda i: (0, 0)),
        out_shape=jax.ShapeDtypeStruct((_B, _TOPK), jnp.int32),
        scratch_shapes=[pltpu.VMEM((_B, _E), jnp.float32)],
        compiler_params=pltpu.CompilerParams(
            dimension_semantics=("arbitrary",),
        ),
    )(x_flat, W_switch, b_switch.reshape(1, _E))


def _moe_body(idx_ref, x_ref, w1_ref, b1_ref, w2_ref, b2_ref, wo_ref, bo_ref,
              o_ref):
    k = pl.program_id(2)
    h = jnp.dot(x_ref[0], w1_ref[0],
                preferred_element_type=jnp.float32,
                precision=jax.lax.Precision.HIGHEST) + b1_ref[0]
    h = _gelu(h)
    h = jnp.dot(h, w2_ref[0],
                preferred_element_type=jnp.float32,
                precision=jax.lax.Precision.HIGHEST) + b2_ref[0]
    h = _gelu(h)
    contrib = jnp.dot(h, wo_ref[...],
                      preferred_element_type=jnp.float32,
                      precision=jax.lax.Precision.HIGHEST)

    @pl.when(k == 0)
    def _set():
        o_ref[0] = contrib + bo_ref[...]

    @pl.when(k != 0)
    def _add():
        o_ref[0] += contrib


def _moe(idx, x, w1, b1, w2, b2, W_out, b_out):
    nsb = _S // _SBLK
    grid = (_B, nsb, _TOPK)

    def e_of(b, k, idx_ref):
        return idx_ref[b * _TOPK + k]

    grid_spec = pltpu.PrefetchScalarGridSpec(
        num_scalar_prefetch=1,
        grid=grid,
        in_specs=[
            pl.BlockSpec((1, _SBLK, _SUB),
                         lambda b, s, k, idx_ref: (b, s, e_of(b, k, idx_ref))),
            pl.BlockSpec((1, _SUB, _SUBH),
                         lambda b, s, k, idx_ref: (e_of(b, k, idx_ref), 0, 0)),
            pl.BlockSpec((1, 1, _SUBH),
                         lambda b, s, k, idx_ref: (e_of(b, k, idx_ref), 0, 0)),
            pl.BlockSpec((1, _SUBH, _SUBH),
                         lambda b, s, k, idx_ref: (e_of(b, k, idx_ref), 0, 0)),
            pl.BlockSpec((1, 1, _SUBH),
                         lambda b, s, k, idx_ref: (e_of(b, k, idx_ref), 0, 0)),
            pl.BlockSpec((_SUBH, _D),
                         lambda b, s, k, idx_ref: (e_of(b, k, idx_ref), 0)),
            pl.BlockSpec((1, _D), lambda b, s, k, idx_ref: (0, 0)),
        ],
        out_specs=pl.BlockSpec((1, _SBLK, _D),
                               lambda b, s, k, idx_ref: (b, s, 0)),
    )
    return pl.pallas_call(
        _moe_body,
        grid_spec=grid_spec,
        out_shape=jax.ShapeDtypeStruct((_B, _S, _D), jnp.float32),
        compiler_params=pltpu.CompilerParams(
            dimension_semantics=("parallel", "parallel", "arbitrary"),
        ),
    )(idx, x, w1, b1.reshape(_E, 1, _SUBH), w2, b2.reshape(_E, 1, _SUBH),
      W_out, b_out.reshape(1, _D))


def kernel(x, W_switch, b_switch, w1, b1, w2, b2, W_out, b_out):
    idx = _route(x.reshape(_B, _S * _D), W_switch, b_switch)
    return _moe(idx.reshape(-1), x, w1, b1, w2, b2, W_out, b_out)


# speedup vs baseline: 1.2524x; 1.2524x over previous
import jax
import jax.numpy as jnp
from jax.experimental import pallas as pl
from jax.experimental.pallas import tpu as pltpu

_B, _S, _D = 4, 2048, 2048
_E = 8
_TOPK = 2
_HID = 8192
_SUB = _D // _E
_SUBH = _HID // _E

_RK = 16384
_SBLK = 512


def _gelu(x):
    return x * 0.5 * (1.0 + jax.lax.erf(x * 0.7071067811865476))


def _router_body(xf_ref, w_ref, bsw_ref, idx_ref, acc_ref):
    i = pl.program_id(0)

    @pl.when(i == 0)
    def _init():
        acc_ref[...] = jnp.zeros_like(acc_ref)

    acc_ref[...] += jnp.dot(
        xf_ref[...], w_ref[...],
        preferred_element_type=jnp.float32,
        precision=jax.lax.Precision.HIGHEST,
    )

    @pl.when(i == pl.num_programs(0) - 1)
    def _finish():
        logits = acc_ref[...] + bsw_ref[...]
        ids = jax.lax.broadcasted_iota(jnp.int32, logits.shape, 1)
        top1 = jnp.argmax(logits, axis=1).astype(jnp.int32)
        masked = jnp.where(ids == top1[:, None], -jnp.inf, logits)
        top2 = jnp.argmax(masked, axis=1).astype(jnp.int32)
        idx_ref[...] = jnp.stack([top1, top2], axis=1)


def _route(x_flat, W_switch, b_switch):
    nsteps = (_S * _D) // _RK
    return pl.pallas_call(
        _router_body,
        grid=(nsteps,),
        in_specs=[
            pl.BlockSpec((_B, _RK), lambda i: (0, i)),
            pl.BlockSpec((_RK, _E), lambda i: (i, 0)),
            pl.BlockSpec((1, _E), lambda i: (0, 0)),
        ],
        out_specs=pl.BlockSpec((_B, _TOPK), lambda i: (0, 0)),
        out_shape=jax.ShapeDtypeStruct((_B, _TOPK), jnp.int32),
        scratch_shapes=[pltpu.VMEM((_B, _E), jnp.float32)],
        compiler_params=pltpu.CompilerParams(
            dimension_semantics=("arbitrary",),
        ),
    )(x_flat, W_switch, b_switch.reshape(1, _E))


def _moe_body(idx_ref, x_ref, w1_ref, b1_ref, w2_ref, b2_ref, wo_ref, bo_ref,
              o_ref):
    k = pl.program_id(2)
    bf = jnp.bfloat16
    h = jnp.dot(x_ref[0].astype(bf), w1_ref[0].astype(bf),
                preferred_element_type=jnp.float32) + b1_ref[0]
    h = _gelu(h)
    h = jnp.dot(h.astype(bf), w2_ref[0].astype(bf),
                preferred_element_type=jnp.float32) + b2_ref[0]
    h = _gelu(h)
    contrib = jnp.dot(h.astype(bf), wo_ref[...].astype(bf),
                      preferred_element_type=jnp.float32)

    @pl.when(k == 0)
    def _set():
        o_ref[0] = contrib + bo_ref[...]

    @pl.when(k != 0)
    def _add():
        o_ref[0] += contrib


def _moe(idx, x, w1, b1, w2, b2, W_out, b_out):
    nsb = _S // _SBLK
    grid = (_B, nsb, _TOPK)

    def e_of(b, k, idx_ref):
        return idx_ref[b * _TOPK + k]

    grid_spec = pltpu.PrefetchScalarGridSpec(
        num_scalar_prefetch=1,
        grid=grid,
        in_specs=[
            pl.BlockSpec((1, _SBLK, _SUB),
                         lambda b, s, k, idx_ref: (b, s, e_of(b, k, idx_ref))),
            pl.BlockSpec((1, _SUB, _SUBH),
                         lambda b, s, k, idx_ref: (e_of(b, k, idx_ref), 0, 0)),
            pl.BlockSpec((1, 1, _SUBH),
                         lambda b, s, k, idx_ref: (e_of(b, k, idx_ref), 0, 0)),
            pl.BlockSpec((1, _SUBH, _SUBH),
                         lambda b, s, k, idx_ref: (e_of(b, k, idx_ref), 0, 0)),
            pl.BlockSpec((1, 1, _SUBH),
                         lambda b, s, k, idx_ref: (e_of(b, k, idx_ref), 0, 0)),
            pl.BlockSpec((_SUBH, _D),
                         lambda b, s, k, idx_ref: (e_of(b, k, idx_ref), 0)),
            pl.BlockSpec((1, _D), lambda b, s, k, idx_ref: (0, 0)),
        ],
        out_specs=pl.BlockSpec((1, _SBLK, _D),
                               lambda b, s, k, idx_ref: (b, s, 0)),
    )
    return pl.pallas_call(
        _moe_body,
        grid_spec=grid_spec,
        out_shape=jax.ShapeDtypeStruct((_B, _S, _D), jnp.float32),
        compiler_params=pltpu.CompilerParams(
            dimension_semantics=("parallel", "parallel", "arbitrary"),
        ),
    )(idx, x, w1, b1.reshape(_E, 1, _SUBH), w2, b2.reshape(_E, 1, _SUBH),
      W_out, b_out.reshape(1, _D))


def kernel(x, W_switch, b_switch, w1, b1, w2, b2, W_out, b_out):
    idx = _route(x.reshape(_B, _S * _D), W_switch, b_switch)
    return _moe(idx.reshape(-1), x, w1, b1, w2, b2, W_out, b_out)


# speedup vs baseline: 7.1580x; 5.7153x over previous
import jax
import jax.numpy as jnp
from jax.experimental import pallas as pl
from jax.experimental.pallas import tpu as pltpu

_B, _S, _D = 4, 2048, 2048
_E = 8
_TOPK = 2
_HID = 8192
_SUB = _D // _E
_SUBH = _HID // _E

_RK = 131072
_SBLK = 512


def _gelu(x):
    return x * 0.5 * (1.0 + jax.lax.erf(x * 0.7071067811865476))


def _router_body(x_ref, wt_ref, bsw_ref, idx_ref, acc_ref):
    i = pl.program_id(0)

    @pl.when(i == 0)
    def _init():
        acc_ref[...] = jnp.zeros_like(acc_ref)

    acc_ref[...] += jax.lax.dot_general(
        x_ref[...].astype(jnp.bfloat16), wt_ref[...],
        (((1,), (1,)), ((), ())),
        preferred_element_type=jnp.float32)

    @pl.when(i == pl.num_programs(0) - 1)
    def _finish():
        logits = acc_ref[...] + bsw_ref[...]
        ids = jax.lax.broadcasted_iota(jnp.int32, logits.shape, 1)
        top1 = jnp.argmax(logits, axis=1).astype(jnp.int32)
        masked = jnp.where(ids == top1[:, None], -jnp.inf, logits)
        top2 = jnp.argmax(masked, axis=1).astype(jnp.int32)
        idx_ref[...] = jnp.stack([top1, top2], axis=1)


def _route(x_flat, Wt, b_switch):
    nsteps = (_S * _D) // _RK
    return pl.pallas_call(
        _router_body,
        grid=(nsteps,),
        in_specs=[
            pl.BlockSpec((_B, _RK), lambda i: (0, i)),
            pl.BlockSpec((_E, _RK), lambda i: (0, i)),
            pl.BlockSpec((1, _E), lambda i: (0, 0)),
        ],
        out_specs=pl.BlockSpec((_B, _TOPK), lambda i: (0, 0)),
        out_shape=jax.ShapeDtypeStruct((_B, _TOPK), jnp.int32),
        scratch_shapes=[pltpu.VMEM((_B, _E), jnp.float32)],
        compiler_params=pltpu.CompilerParams(
            dimension_semantics=("arbitrary",),
        ),
    )(x_flat, Wt, b_switch.reshape(1, _E))


def _moe_body(idx_ref, x_ref, w1_ref, b1_ref, w2_ref, b2_ref, wo_ref, bo_ref,
              o_ref):
    k = pl.program_id(2)
    bf = jnp.bfloat16
    h = jnp.dot(x_ref[0].astype(bf), w1_ref[0].astype(bf),
                preferred_element_type=jnp.float32) + b1_ref[0]
    h = _gelu(h)
    h = jnp.dot(h.astype(bf), w2_ref[0].astype(bf),
                preferred_element_type=jnp.float32) + b2_ref[0]
    h = _gelu(h)
    contrib = jnp.dot(h.astype(bf), wo_ref[...].astype(bf),
                      preferred_element_type=jnp.float32)

    @pl.when(k == 0)
    def _set():
        o_ref[0] = contrib + bo_ref[...]

    @pl.when(k != 0)
    def _add():
        o_ref[0] += contrib


def _moe(idx, x, w1, b1, w2, b2, W_out, b_out):
    nsb = _S // _SBLK
    grid = (_B, nsb, _TOPK)

    def e_of(b, k, idx_ref):
        return idx_ref[b * _TOPK + k]

    grid_spec = pltpu.PrefetchScalarGridSpec(
        num_scalar_prefetch=1,
        grid=grid,
        in_specs=[
            pl.BlockSpec((1, _SBLK, _SUB),
                         lambda b, s, k, idx_ref: (b, s, e_of(b, k, idx_ref))),
            pl.BlockSpec((1, _SUB, _SUBH),
                         lambda b, s, k, idx_ref: (e_of(b, k, idx_ref), 0, 0)),
            pl.BlockSpec((1, 1, _SUBH),
                         lambda b, s, k, idx_ref: (e_of(b, k, idx_ref), 0, 0)),
            pl.BlockSpec((1, _SUBH, _SUBH),
                         lambda b, s, k, idx_ref: (e_of(b, k, idx_ref), 0, 0)),
            pl.BlockSpec((1, 1, _SUBH),
                         lambda b, s, k, idx_ref: (e_of(b, k, idx_ref), 0, 0)),
            pl.BlockSpec((_SUBH, _D),
                         lambda b, s, k, idx_ref: (e_of(b, k, idx_ref), 0)),
            pl.BlockSpec((1, _D), lambda b, s, k, idx_ref: (0, 0)),
        ],
        out_specs=pl.BlockSpec((1, _SBLK, _D),
                               lambda b, s, k, idx_ref: (b, s, 0)),
    )
    return pl.pallas_call(
        _moe_body,
        grid_spec=grid_spec,
        out_shape=jax.ShapeDtypeStruct((_B, _S, _D), jnp.float32),
        compiler_params=pltpu.CompilerParams(
            dimension_semantics=("parallel", "parallel", "arbitrary"),
        ),
    )(idx, x, w1, b1.reshape(_E, 1, _SUBH), w2, b2.reshape(_E, 1, _SUBH),
      W_out, b_out.reshape(1, _D))


def kernel(x, W_switch, b_switch, w1, b1, w2, b2, W_out, b_out):
    idx = _route(x.reshape(_B, _S * _D),
                 W_switch.T.astype(jnp.bfloat16), b_switch)
    return _moe(idx.reshape(-1), x, w1, b1, w2, b2, W_out, b_out)


# speedup vs baseline: 8.1588x; 1.1398x over previous
import jax
import jax.numpy as jnp
from jax.experimental import pallas as pl
from jax.experimental.pallas import tpu as pltpu

_B, _S, _D = 4, 2048, 2048
_E = 8
_TOPK = 2
_HID = 8192
_SUB = _D // _E
_SUBH = _HID // _E

_RK = 131072
_SBLK = 1024


def _gelu(x):
    return x * 0.5 * (1.0 + jax.lax.erf(x * 0.7071067811865476))


def _router_body(x_ref, wt_ref, bsw_ref, idx_ref, acc_ref):
    i = pl.program_id(0)

    @pl.when(i == 0)
    def _init():
        acc_ref[...] = jnp.zeros_like(acc_ref)

    acc_ref[...] += jax.lax.dot_general(
        x_ref[...], wt_ref[...],
        (((1,), (1,)), ((), ())),
        preferred_element_type=jnp.float32,
        precision=jax.lax.Precision.HIGHEST)

    @pl.when(i == pl.num_programs(0) - 1)
    def _finish():
        logits = acc_ref[...] + bsw_ref[...]
        m = jnp.max(logits, axis=1, keepdims=True)
        p = jnp.exp(logits - m)
        p = p / jnp.sum(p, axis=1, keepdims=True)
        ids = jax.lax.broadcasted_iota(jnp.int32, p.shape, 1)
        v1 = jnp.max(p, axis=1, keepdims=True)
        top1 = jnp.min(jnp.where(p == v1, ids, _E), axis=1).astype(jnp.int32)
        p2 = jnp.where(ids == top1[:, None], -1.0, p)
        v2 = jnp.max(p2, axis=1, keepdims=True)
        top2 = jnp.min(jnp.where(p2 == v2, ids, _E), axis=1).astype(jnp.int32)
        idx_ref[...] = jnp.stack([top1, top2], axis=1)


def _route(x_flat, Wt, b_switch):
    nsteps = (_S * _D) // _RK
    return pl.pallas_call(
        _router_body,
        grid=(nsteps,),
        in_specs=[
            pl.BlockSpec((_B, _RK), lambda i: (0, i)),
            pl.BlockSpec((_E, _RK), lambda i: (0, i)),
            pl.BlockSpec((1, _E), lambda i: (0, 0)),
        ],
        out_specs=pl.BlockSpec((_B, _TOPK), lambda i: (0, 0)),
        out_shape=jax.ShapeDtypeStruct((_B, _TOPK), jnp.int32),
        scratch_shapes=[pltpu.VMEM((_B, _E), jnp.float32)],
        compiler_params=pltpu.CompilerParams(
            dimension_semantics=("arbitrary",),
        ),
    )(x_flat, Wt, b_switch.reshape(1, _E))


def _moe_body(idx_ref, x_ref, w1_ref, b1_ref, w2_ref, b2_ref, wo_ref, bo_ref,
              o_ref):
    k = pl.program_id(2)
    bf = jnp.bfloat16
    h = jnp.dot(x_ref[0].astype(bf), w1_ref[0].astype(bf),
                preferred_element_type=jnp.float32) + b1_ref[0]
    h = _gelu(h.astype(bf))
    h = jnp.dot(h, w2_ref[0].astype(bf),
                preferred_element_type=jnp.float32) + b2_ref[0]
    h = _gelu(h.astype(bf))
    contrib = jnp.dot(h, wo_ref[...].astype(bf),
                      preferred_element_type=jnp.float32)

    @pl.when(k == 0)
    def _set():
        o_ref[0] = contrib + bo_ref[...]

    @pl.when(k != 0)
    def _add():
        o_ref[0] += contrib


def _moe(idx, x, w1, b1, w2, b2, W_out, b_out):
    nsb = _S // _SBLK
    grid = (_B, nsb, _TOPK)

    def e_of(b, k, idx_ref):
        return idx_ref[b * _TOPK + k]

    grid_spec = pltpu.PrefetchScalarGridSpec(
        num_scalar_prefetch=1,
        grid=grid,
        in_specs=[
            pl.BlockSpec((1, _SBLK, _SUB),
                         lambda b, s, k, idx_ref: (b, s, e_of(b, k, idx_ref))),
            pl.BlockSpec((1, _SUB, _SUBH),
                         lambda b, s, k, idx_ref: (e_of(b, k, idx_ref), 0, 0)),
            pl.BlockSpec((1, 1, _SUBH),
                         lambda b, s, k, idx_ref: (e_of(b, k, idx_ref), 0, 0)),
            pl.BlockSpec((1, _SUBH, _SUBH),
                         lambda b, s, k, idx_ref: (e_of(b, k, idx_ref), 0, 0)),
            pl.BlockSpec((1, 1, _SUBH),
                         lambda b, s, k, idx_ref: (e_of(b, k, idx_ref), 0, 0)),
            pl.BlockSpec((_SUBH, _D),
                         lambda b, s, k, idx_ref: (e_of(b, k, idx_ref), 0)),
            pl.BlockSpec((1, _D), lambda b, s, k, idx_ref: (0, 0)),
        ],
        out_specs=pl.BlockSpec((1, _SBLK, _D),
                               lambda b, s, k, idx_ref: (b, s, 0)),
    )
    return pl.pallas_call(
        _moe_body,
        grid_spec=grid_spec,
        out_shape=jax.ShapeDtypeStruct((_B, _S, _D), jnp.float32),
        compiler_params=pltpu.CompilerParams(
            dimension_semantics=("parallel", "parallel", "arbitrary"),
        ),
    )(idx, x, w1, b1.reshape(_E, 1, _SUBH), w2, b2.reshape(_E, 1, _SUBH),
      W_out, b_out.reshape(1, _D))


def kernel(x, W_switch, b_switch, w1, b1, w2, b2, W_out, b_out):
    idx = _route(x.reshape(_B, _S * _D), W_switch.T, b_switch)
    return _moe(idx.reshape(-1), x, w1, b1, w2, b2, W_out, b_out)
